# baseline (speedup 1.0000x reference)
import functools

import jax
import jax.numpy as jnp
from jax import lax
from jax.experimental import pallas as pl
from jax.experimental.pallas import tpu as pltpu
from jax.experimental.pallas import tpu_sc as plsc

NN = 10000
NE = 320000
D = 128
D2 = D // 2
LH = 64
G4 = 4 * LH

NC = 2
NS = 16
L = 16
NW = NC * NS
CH = 128
DNCH = 80
DEPW = DNCH * CH
ANCH = 158
AEPW = ANCH * CH
NNP = 10240
NPS = 624
NREM = NN - NS * NPS


def _sc_deg_body(col_hbm, ea_hbm, pdeg_hbm, colv, eav, degv):
    cid = lax.axis_index("c")
    sid = lax.axis_index("s")
    wid = sid * NC + cid
    pltpu.sync_copy(col_hbm.at[wid], colv)
    pltpu.sync_copy(ea_hbm.at[wid], eav)

    def zero(i, _):
        degv[pl.ds(i * L, L)] = jnp.zeros((L,), jnp.float32)
        return 0

    lax.fori_loop(0, NNP // L, zero, 0)

    NPC = CH // L

    def body(i, _):
        r = i // NPC
        q = i % NPC
        idx = colv[r, pl.ds(q * L, L)]
        vals = eav[r, pl.ds(q * L, L)]
        plsc.addupdate_scatter(degv, [idx], vals)
        return 0

    lax.fori_loop(0, DNCH * NPC, body, 0)
    pltpu.sync_copy(degv, pdeg_hbm.at[pl.ds(wid * NNP, NNP)])


def _sc_agg_body(y_hbm, row_hbm, col_hbm, ea_hbm, out_hbm, rowv, colv, eav,
                 bufa, bufb, aggs, sem, sem2):
    cid = lax.axis_index("c")
    sid = lax.axis_index("s")
    pltpu.sync_copy(row_hbm.at[sid], rowv)
    pltpu.sync_copy(col_hbm.at[sid], colv)
    pltpu.sync_copy(ea_hbm.at[sid], eav)
    yh = y_hbm.at[cid]

    def zbuf(i, _):
        r = i // (D2 // L)
        q = i % (D2 // L)
        bufa[r, pl.ds(q * L, L)] = jnp.zeros((L,), jnp.float32)
        return 0

    lax.fori_loop(0, CH * (D2 // L), zbuf, 0)

    NZF = NPS // CH
    NZR = NPS % CH

    def zagg(k, _):
        pltpu.sync_copy(bufa, aggs.at[pl.ds(sid * NPS + k * CH, CH)])
        return 0

    lax.fori_loop(0, NZF, zagg, 0)
    pltpu.sync_copy(bufa.at[pl.ds(0, NZR)],
                    aggs.at[pl.ds(sid * NPS + NZF * CH, NZR)])

    @pl.when(sid == NS - 1)
    def _():
        pltpu.sync_copy(bufa.at[pl.ds(0, NREM)],
                        aggs.at[pl.ds(NS * NPS, NREM)])

    plsc.subcore_barrier()

    def _scale(buf, j):
        def sgroup(g0, _):
            ev = eav[j, pl.ds(g0 * L, L)]
            svs = [jnp.full((L,), ev[t], jnp.float32) for t in range(L)]
            for t in range(L):
                r = g0 * L + t
                for q in range(D2 // L):
                    buf[r, pl.ds(q * L, L)] = buf[r, pl.ds(q * L, L)] * svs[t]
            return 0

        lax.fori_loop(0, CH // L, sgroup, 0)

    def _gather(buf, j, sm):
        pltpu.async_copy(yh.at[rowv.at[j]], buf, sm)

    def _gwait(buf, j, sm):
        pltpu.make_async_copy(yh.at[rowv.at[j]], buf, sm).wait()

    NH = ANCH // 2
    _gather(bufa, 0, sem)

    def chunk_body(k, _):
        j0 = 2 * k
        j1 = j0 + 1
        _gather(bufb, j1, sem2)
        _gwait(bufa, j0, sem)
        _scale(bufa, j0)
        pltpu.sync_copy(bufa, aggs.at[colv.at[j0]], add=True)

        @pl.when(k < NH - 1)
        def _():
            _gather(bufa, j0 + 2, sem)

        _gwait(bufb, j1, sem2)
        _scale(bufb, j1)
        pltpu.sync_copy(bufb, aggs.at[colv.at[j1]], add=True)
        return 0

    lax.fori_loop(0, NH, chunk_body, 0)
    plsc.subcore_barrier()
    pltpu.sync_copy(aggs.at[pl.ds(sid * NPS, NPS)],
                    out_hbm.at[pl.ds(cid * NN + sid * NPS, NPS)])

    @pl.when(sid == NS - 1)
    def _():
        pltpu.sync_copy(aggs.at[pl.ds(NS * NPS, NREM)],
                        out_hbm.at[pl.ds(cid * NN + NS * NPS, NREM)])


@functools.lru_cache(maxsize=1)
def _sc_kernels():
    mesh = plsc.VectorSubcoreMesh(core_axis_name="c", subcore_axis_name="s",
                                  num_cores=NC, num_subcores=NS)
    params = pltpu.CompilerParams(needs_layout_passes=False)
    sc_deg = pl.kernel(
        _sc_deg_body,
        out_type=jax.ShapeDtypeStruct((NW * NNP,), jnp.float32),
        mesh=mesh,
        compiler_params=params,
        scratch_types=[
            pltpu.VMEM((DNCH, CH), jnp.int32),
            pltpu.VMEM((DNCH, CH), jnp.float32),
            pltpu.VMEM((NNP,), jnp.float32),
        ],
    )
    sc_agg = pl.kernel(
        _sc_agg_body,
        out_type=jax.ShapeDtypeStruct((NC * NN, D2), jnp.float32),
        mesh=mesh,
        compiler_params=pltpu.CompilerParams(needs_layout_passes=False,
                                             use_tc_tiling_on_sc=False),
        scratch_types=[
            pltpu.VMEM((ANCH, CH), jnp.int32),
            pltpu.VMEM((ANCH, CH), jnp.int32),
            pltpu.VMEM((ANCH, CH), jnp.float32),
            pltpu.VMEM((CH, D2), jnp.float32),
            pltpu.VMEM((CH, D2), jnp.float32),
            pltpu.VMEM_SHARED((NN, D2), jnp.float32),
            pltpu.SemaphoreType.DMA,
            pltpu.SemaphoreType.DMA,
        ],
    )
    return sc_deg, sc_agg


def _tc_dinv_body(pdeg_ref, dinv_ref):
    deg = jnp.sum(pdeg_ref[...], axis=0, keepdims=True)
    safe = jnp.where(deg > 0, deg, 1.0)
    dinv_ref[...] = jnp.where(deg > 0, lax.rsqrt(safe), 0.0)


_tc_dinv = pl.pallas_call(
    _tc_dinv_body, out_shape=jax.ShapeDtypeStruct((1, NNP), jnp.float32))


def _tc_y1_body(x_ref, dinv_ref, w_ref, y_ref):
    y_ref[...] = jnp.dot(x_ref[...] * dinv_ref[...], w_ref[...],
                         preferred_element_type=jnp.float32)


_tc_y1 = pl.pallas_call(
    _tc_y1_body, out_shape=jax.ShapeDtypeStruct((NN, D), jnp.float32))


def _combine(agg_ref):
    return jnp.concatenate([agg_ref[0:NN, :], agg_ref[NN:2 * NN, :]], axis=1)


def _tc_mid_body(agg_ref, dinv_ref, b1_ref, w2_ref, y2_ref):
    h1 = jax.nn.relu(_combine(agg_ref) * dinv_ref[...] + b1_ref[...])
    y2_ref[...] = jnp.dot(h1 * dinv_ref[...], w2_ref[...],
                          preferred_element_type=jnp.float32)


_tc_mid = pl.pallas_call(
    _tc_mid_body, out_shape=jax.ShapeDtypeStruct((NN, D), jnp.float32))


def _tc_h2_body(agg_ref, dinv_ref, b2_ref, h2_ref):
    h2_ref[...] = jax.nn.relu(_combine(agg_ref) * dinv_ref[...] + b2_ref[...])


_tc_h2 = pl.pallas_call(
    _tc_h2_body, out_shape=jax.ShapeDtypeStruct((NN, D), jnp.float32))


def _tc_lstm_body(h2_ref, wf_ref, wb_ref, wr_ref, bc_ref,
                  wl_ref, bl_ref, out_ref, gf_s, gb_s, hcat):
    h2 = h2_ref[...]
    gf_s[...] = jnp.dot(h2, wf_ref[...],
                        preferred_element_type=jnp.float32) + bc_ref[...]
    gb_s[...] = jnp.dot(h2, wb_ref[...], preferred_element_type=jnp.float32)

    def step(t, carry):
        h, c = carry
        g = (gf_s[pl.ds(t, 1), :] + gb_s[pl.ds(NN - 1 - t, 1), :] +
             jnp.dot(h, wr_ref[...], preferred_element_type=jnp.float32))
        sg = jax.nn.sigmoid(g[:, 0:384])
        th = jnp.tanh(g[:, 384:512])
        c = sg[:, 128:256] * c + sg[:, 0:128] * th
        h = sg[:, 256:384] * jnp.tanh(c)
        hcat[pl.ds(t, 1), 0:LH] = h[:, 0:LH]
        hcat[pl.ds(NN - 1 - t, 1), LH:2 * LH] = h[:, LH:2 * LH]
        return h, c

    z = jnp.zeros((1, 2 * LH), jnp.float32)
    lax.fori_loop(0, NN, step, (z, z))
    p = jnp.dot(hcat[...], wl_ref[...],
                preferred_element_type=jnp.float32) + bl_ref[...]
    out_ref[...] = jax.nn.sigmoid(p)


_tc_lstm = pl.pallas_call(
    _tc_lstm_body,
    out_shape=jax.ShapeDtypeStruct((NN, 1), jnp.float32),
    scratch_shapes=[
        pltpu.VMEM((NN, 512), jnp.float32),
        pltpu.VMEM((NN, 512), jnp.float32),
        pltpu.VMEM((NN, 2 * LH), jnp.float32),
    ],
    compiler_params=pltpu.CompilerParams(vmem_limit_bytes=62 * 1024 * 1024),
)


def _lstm_weights(Wih_f, Whh_f, bih_f, bhh_f, Wih_b, Whh_b, bih_b, bhh_b):
    def slots(WT):
        return WT[:, 0:LH], WT[:, LH:2 * LH], WT[:, 3 * LH:4 * LH], WT[:, 2 * LH:3 * LH]

    def place(WT, off):
        i_, f_, o_, g_ = slots(WT)
        out = jnp.zeros((WT.shape[0], 512), WT.dtype)
        out = out.at[:, 0 + off:LH + off].set(i_)
        out = out.at[:, 128 + off:128 + LH + off].set(f_)
        out = out.at[:, 256 + off:256 + LH + off].set(o_)
        out = out.at[:, 384 + off:384 + LH + off].set(g_)
        return out

    WF = place(Wih_f.T, 0)
    WB = place(Wih_b.T, LH)
    WR = jnp.zeros((2 * LH, 512), Wih_f.dtype)
    WR = WR + place(jnp.concatenate([Whh_f.T, jnp.zeros_like(Whh_f.T)], 0), 0)
    WR = WR + place(jnp.concatenate([jnp.zeros_like(Whh_b.T), Whh_b.T], 0), LH)
    bc = (place((bih_f + bhh_f).reshape(1, G4), 0) +
          place((bih_b + bhh_b).reshape(1, G4), LH))
    return WF, WB, WR, bc


def kernel(x, edge_index, edge_attr, W1, b1, W2, b2, Wih_f, Whh_f, bih_f,
           bhh_f, Wih_b, Whh_b, bih_b, bhh_b, Wl, bl):
    _sc_deg, _sc_agg = _sc_kernels()
    row = edge_index[0]
    col = edge_index[1]
    dpad = NW * DEPW - NE
    colp_d = jnp.concatenate([col, jnp.zeros((dpad,), jnp.int32)])
    colp_d = colp_d.reshape(NW, DNCH, CH)
    eap_d = jnp.concatenate([edge_attr, jnp.zeros((dpad,), jnp.float32)])
    eap_d = eap_d.reshape(NW, DNCH, CH)

    apad = NS * AEPW - NE
    zi = jnp.zeros((apad,), jnp.int32)
    rowp = jnp.concatenate([row, zi]).reshape(NS, ANCH, CH)
    colp = jnp.concatenate([col, zi]).reshape(NS, ANCH, CH)
    eap = jnp.concatenate([edge_attr, jnp.zeros((apad,), jnp.float32)])
    eap = eap.reshape(NS, ANCH, CH)

    pdeg = _sc_deg(colp_d, eap_d).reshape(NW, NNP)
    dinv = _tc_dinv(pdeg)[:, :NN].reshape(NN, 1)

    y1 = _tc_y1(x, dinv, W1)
    y1s = jnp.stack([y1[:, 0:D2], y1[:, D2:D]])
    agg1 = _sc_agg(y1s, rowp, colp, eap)
    y2 = _tc_mid(agg1, dinv, b1.reshape(1, D), W2)
    y2s = jnp.stack([y2[:, 0:D2], y2[:, D2:D]])
    agg2 = _sc_agg(y2s, rowp, colp, eap)

    WF, WB, WR, bc = _lstm_weights(Wih_f, Whh_f, bih_f, bhh_f, Wih_b, Whh_b,
                                   bih_b, bhh_b)
    h2 = _tc_h2(agg2, dinv, b2.reshape(1, D))
    out = _tc_lstm(h2, WF, WB, WR, bc, Wl, bl.reshape(1, 1))
    return out.reshape(1, NN)


# speedup vs baseline: 1.1805x; 1.1805x over previous
import functools

import jax
import jax.numpy as jnp
from jax import lax
from jax.experimental import pallas as pl
from jax.experimental.pallas import tpu as pltpu
from jax.experimental.pallas import tpu_sc as plsc

NN = 10000
NE = 320000
D = 128
D2 = D // 2
LH = 64
G4 = 4 * LH

NC = 2
NS = 16
L = 16
NW = NC * NS
CH = 128
DNCH = 80
DEPW = DNCH * CH
SUB = 32
NBUF = 4
NSLOT = 316
AEPW = NSLOT * SUB
NNP = 10240
NPS = 624
NREM = NN - NS * NPS


def _sc_deg_body(col_hbm, ea_hbm, pdeg_hbm, colv, eav, degv):
    cid = lax.axis_index("c")
    sid = lax.axis_index("s")
    wid = sid * NC + cid
    pltpu.sync_copy(col_hbm.at[wid], colv)
    pltpu.sync_copy(ea_hbm.at[wid], eav)

    def zero(i, _):
        degv[pl.ds(i * L, L)] = jnp.zeros((L,), jnp.float32)
        return 0

    lax.fori_loop(0, NNP // L, zero, 0)

    NPC = CH // L

    def body(i, _):
        r = i // NPC
        q = i % NPC
        idx = colv[r, pl.ds(q * L, L)]
        vals = eav[r, pl.ds(q * L, L)]
        plsc.addupdate_scatter(degv, [idx], vals)
        return 0

    lax.fori_loop(0, DNCH * NPC, body, 0)
    pltpu.sync_copy(degv, pdeg_hbm.at[pl.ds(wid * NNP, NNP)])


def _sc_agg_body(y_hbm, row_hbm, col_hbm, ea_hbm, out_hbm, rowv, colv, eav,
                 bufs, aggs, gsems, ssems):
    cid = lax.axis_index("c")
    sid = lax.axis_index("s")
    wid = sid * NC + cid
    pltpu.sync_copy(row_hbm.at[wid], rowv)
    pltpu.sync_copy(col_hbm.at[wid], colv)
    pltpu.sync_copy(ea_hbm.at[wid], eav)

    def zbuf(i, _):
        r = i // (D // L)
        q = i % (D // L)
        bufs[0][r, pl.ds(q * L, L)] = jnp.zeros((L,), jnp.float32)
        return 0

    lax.fori_loop(0, SUB * (D // L), zbuf, 0)

    NZF = NPS // SUB
    NZR = NPS % SUB

    def zagg(k, _):
        pltpu.sync_copy(bufs[0], aggs.at[pl.ds(sid * NPS + k * SUB, SUB)])
        return 0

    lax.fori_loop(0, NZF, zagg, 0)
    pltpu.sync_copy(bufs[0].at[pl.ds(0, NZR)],
                    aggs.at[pl.ds(sid * NPS + NZF * SUB, NZR)])

    @pl.when(sid == NS - 1)
    def _():
        pltpu.sync_copy(bufs[0].at[pl.ds(0, NREM)],
                        aggs.at[pl.ds(NS * NPS, NREM)])

    plsc.subcore_barrier()

    def _scale(buf, j):
        def sgroup(g0, _):
            ev = eav[j, pl.ds(g0 * L, L)]
            svs = [jnp.full((L,), ev[t], jnp.float32) for t in range(L)]
            for t in range(L):
                r = g0 * L + t
                for q in range(D // L):
                    buf[r, pl.ds(q * L, L)] = buf[r, pl.ds(q * L, L)] * svs[t]
            return 0

        lax.fori_loop(0, SUB // L, sgroup, 0)

    def _gather(b, j):
        pltpu.async_copy(y_hbm.at[rowv.at[j]], bufs[b], gsems[b])

    def _gwait(b, j):
        pltpu.make_async_copy(y_hbm.at[rowv.at[j]], bufs[b], gsems[b]).wait()

    def _scat(b, j):
        pltpu.async_copy(bufs[b], aggs.at[colv.at[j]], ssems[b], add=True)

    def _swait(b, j):
        pltpu.make_async_copy(bufs[b], aggs.at[colv.at[j]], ssems[b]).wait()

    for b in range(2):
        _gather(b, b)

    def slot_body(k, _):
        j0 = 4 * k
        for b in range(NBUF):
            j = j0 + b

            @pl.when(j >= 2)
            def _():
                _swait((b - 2) % NBUF, j - 2)

            @pl.when(j + 2 < NSLOT)
            def _():
                _gather((b + 2) % NBUF, j + 2)

            _gwait(b, j)
            _scale(bufs[b], j)
            _scat(b, j)
        return 0

    lax.fori_loop(0, NSLOT // NBUF, slot_body, 0)
    for j in range(NSLOT - 2, NSLOT):
        _swait(j % NBUF, j)
    plsc.subcore_barrier()
    pltpu.sync_copy(aggs.at[pl.ds(sid * NPS, NPS)],
                    out_hbm.at[pl.ds(cid * NN + sid * NPS, NPS)])

    @pl.when(sid == NS - 1)
    def _():
        pltpu.sync_copy(aggs.at[pl.ds(NS * NPS, NREM)],
                        out_hbm.at[pl.ds(cid * NN + NS * NPS, NREM)])


@functools.lru_cache(maxsize=1)
def _sc_kernels():
    mesh = plsc.VectorSubcoreMesh(core_axis_name="c", subcore_axis_name="s",
                                  num_cores=NC, num_subcores=NS)
    params = pltpu.CompilerParams(needs_layout_passes=False)
    sc_deg = pl.kernel(
        _sc_deg_body,
        out_type=jax.ShapeDtypeStruct((NW * NNP,), jnp.float32),
        mesh=mesh,
        compiler_params=params,
        scratch_types=[
            pltpu.VMEM((DNCH, CH), jnp.int32),
            pltpu.VMEM((DNCH, CH), jnp.float32),
            pltpu.VMEM((NNP,), jnp.float32),
        ],
    )
    sc_agg = pl.kernel(
        _sc_agg_body,
        out_type=jax.ShapeDtypeStruct((NC * NN, D), jnp.float32),
        mesh=mesh,
        compiler_params=pltpu.CompilerParams(needs_layout_passes=False,
                                             use_tc_tiling_on_sc=False),
        scratch_types=[
            pltpu.VMEM((NSLOT, SUB), jnp.int32),
            pltpu.VMEM((NSLOT, SUB), jnp.int32),
            pltpu.VMEM((NSLOT, SUB), jnp.float32),
            tuple(pltpu.VMEM((SUB, D), jnp.float32) for _ in range(NBUF)),
            pltpu.VMEM_SHARED((NN, D), jnp.float32),
            tuple(pltpu.SemaphoreType.DMA for _ in range(NBUF)),
            tuple(pltpu.SemaphoreType.DMA for _ in range(NBUF)),
        ],
    )
    return sc_deg, sc_agg


def _tc_dinv_body(pdeg_ref, dinv_ref):
    deg = jnp.sum(pdeg_ref[...], axis=0, keepdims=True)
    safe = jnp.where(deg > 0, deg, 1.0)
    dinv_ref[...] = jnp.where(deg > 0, lax.rsqrt(safe), 0.0)


_tc_dinv = pl.pallas_call(
    _tc_dinv_body, out_shape=jax.ShapeDtypeStruct((1, NNP), jnp.float32))


def _tc_y1_body(x_ref, dinv_ref, w_ref, y_ref):
    y_ref[...] = jnp.dot(x_ref[...] * dinv_ref[...], w_ref[...],
                         preferred_element_type=jnp.float32)


_tc_y1 = pl.pallas_call(
    _tc_y1_body, out_shape=jax.ShapeDtypeStruct((NN, D), jnp.float32))


def _combine(agg_ref):
    return agg_ref[0:NN, :] + agg_ref[NN:2 * NN, :]


def _tc_mid_body(agg_ref, dinv_ref, b1_ref, w2_ref, y2_ref):
    h1 = jax.nn.relu(_combine(agg_ref) * dinv_ref[...] + b1_ref[...])
    y2_ref[...] = jnp.dot(h1 * dinv_ref[...], w2_ref[...],
                          preferred_element_type=jnp.float32)


_tc_mid = pl.pallas_call(
    _tc_mid_body, out_shape=jax.ShapeDtypeStruct((NN, D), jnp.float32))


def _tc_h2_body(agg_ref, dinv_ref, b2_ref, h2_ref):
    h2_ref[...] = jax.nn.relu(_combine(agg_ref) * dinv_ref[...] + b2_ref[...])


_tc_h2 = pl.pallas_call(
    _tc_h2_body, out_shape=jax.ShapeDtypeStruct((NN, D), jnp.float32))


def _tc_lstm_body(h2_ref, wf_ref, wb_ref, wr_ref, bc_ref,
                  wl_ref, bl_ref, out_ref, gf_s, gb_s, hcat):
    h2 = h2_ref[...]
    gf_s[...] = jnp.dot(h2, wf_ref[...],
                        preferred_element_type=jnp.float32) + bc_ref[...]
    gb_s[...] = jnp.dot(h2, wb_ref[...], preferred_element_type=jnp.float32)

    def step(t, carry):
        h, c = carry
        g = (gf_s[pl.ds(t, 1), :] + gb_s[pl.ds(NN - 1 - t, 1), :] +
             jnp.dot(h, wr_ref[...], preferred_element_type=jnp.float32))
        sg = jax.nn.sigmoid(g[:, 0:384])
        th = jnp.tanh(g[:, 384:512])
        c = sg[:, 128:256] * c + sg[:, 0:128] * th
        h = sg[:, 256:384] * jnp.tanh(c)
        hcat[pl.ds(t, 1), 0:LH] = h[:, 0:LH]
        hcat[pl.ds(NN - 1 - t, 1), LH:2 * LH] = h[:, LH:2 * LH]
        return h, c

    z = jnp.zeros((1, 2 * LH), jnp.float32)
    lax.fori_loop(0, NN, step, (z, z))
    p = jnp.dot(hcat[...], wl_ref[...],
                preferred_element_type=jnp.float32) + bl_ref[...]
    out_ref[...] = jax.nn.sigmoid(p)


_tc_lstm = pl.pallas_call(
    _tc_lstm_body,
    out_shape=jax.ShapeDtypeStruct((NN, 1), jnp.float32),
    scratch_shapes=[
        pltpu.VMEM((NN, 512), jnp.float32),
        pltpu.VMEM((NN, 512), jnp.float32),
        pltpu.VMEM((NN, 2 * LH), jnp.float32),
    ],
    compiler_params=pltpu.CompilerParams(vmem_limit_bytes=62 * 1024 * 1024),
)


def _lstm_weights(Wih_f, Whh_f, bih_f, bhh_f, Wih_b, Whh_b, bih_b, bhh_b):
    def slots(WT):
        return WT[:, 0:LH], WT[:, LH:2 * LH], WT[:, 3 * LH:4 * LH], WT[:, 2 * LH:3 * LH]

    def place(WT, off):
        i_, f_, o_, g_ = slots(WT)
        out = jnp.zeros((WT.shape[0], 512), WT.dtype)
        out = out.at[:, 0 + off:LH + off].set(i_)
        out = out.at[:, 128 + off:128 + LH + off].set(f_)
        out = out.at[:, 256 + off:256 + LH + off].set(o_)
        out = out.at[:, 384 + off:384 + LH + off].set(g_)
        return out

    WF = place(Wih_f.T, 0)
    WB = place(Wih_b.T, LH)
    WR = jnp.zeros((2 * LH, 512), Wih_f.dtype)
    WR = WR + place(jnp.concatenate([Whh_f.T, jnp.zeros_like(Whh_f.T)], 0), 0)
    WR = WR + place(jnp.concatenate([jnp.zeros_like(Whh_b.T), Whh_b.T], 0), LH)
    bc = (place((bih_f + bhh_f).reshape(1, G4), 0) +
          place((bih_b + bhh_b).reshape(1, G4), LH))
    return WF, WB, WR, bc


def kernel(x, edge_index, edge_attr, W1, b1, W2, b2, Wih_f, Whh_f, bih_f,
           bhh_f, Wih_b, Whh_b, bih_b, bhh_b, Wl, bl):
    _sc_deg, _sc_agg = _sc_kernels()
    row = edge_index[0]
    col = edge_index[1]
    dpad = NW * DEPW - NE
    colp_d = jnp.concatenate([col, jnp.zeros((dpad,), jnp.int32)])
    colp_d = colp_d.reshape(NW, DNCH, CH)
    eap_d = jnp.concatenate([edge_attr, jnp.zeros((dpad,), jnp.float32)])
    eap_d = eap_d.reshape(NW, DNCH, CH)

    apad = NW * AEPW - NE
    zi = jnp.zeros((apad,), jnp.int32)
    rowp = jnp.concatenate([row, zi]).reshape(NW, NSLOT, SUB)
    colp = jnp.concatenate([col, zi]).reshape(NW, NSLOT, SUB)
    eap = jnp.concatenate([edge_attr, jnp.zeros((apad,), jnp.float32)])
    eap = eap.reshape(NW, NSLOT, SUB)

    pdeg = _sc_deg(colp_d, eap_d).reshape(NW, NNP)
    dinv = _tc_dinv(pdeg)[:, :NN].reshape(NN, 1)

    y1 = _tc_y1(x, dinv, W1)
    agg1 = _sc_agg(y1, rowp, colp, eap)
    y2 = _tc_mid(agg1, dinv, b1.reshape(1, D), W2)
    agg2 = _sc_agg(y2, rowp, colp, eap)

    WF, WB, WR, bc = _lstm_weights(Wih_f, Whh_f, bih_f, bhh_f, Wih_b, Whh_b,
                                   bih_b, bhh_b)
    h2 = _tc_h2(agg2, dinv, b2.reshape(1, D))
    out = _tc_lstm(h2, WF, WB, WR, bc, Wl, bl.reshape(1, 1))
    return out.reshape(1, NN)


# speedup vs baseline: 1.1831x; 1.0022x over previous
import functools

import jax
import jax.numpy as jnp
from jax import lax
from jax.experimental import pallas as pl
from jax.experimental.pallas import tpu as pltpu
from jax.experimental.pallas import tpu_sc as plsc

NN = 10000
NE = 320000
D = 128
D2 = D // 2
LH = 64
G4 = 4 * LH

NC = 2
NS = 16
L = 16
NW = NC * NS
CH = 128
DNCH = 80
DEPW = DNCH * CH
SUB = 32
NBUF = 4
NSLOT = 316
AEPW = NSLOT * SUB
NNP = 10240
NPS = 624
NREM = NN - NS * NPS


def _sc_deg_body(col_hbm, ea_hbm, pdeg_hbm, colv, eav, degv):
    cid = lax.axis_index("c")
    sid = lax.axis_index("s")
    wid = sid * NC + cid
    pltpu.sync_copy(col_hbm.at[wid], colv)
    pltpu.sync_copy(ea_hbm.at[wid], eav)

    def zero(i, _):
        degv[pl.ds(i * L, L)] = jnp.zeros((L,), jnp.float32)
        return 0

    lax.fori_loop(0, NNP // L, zero, 0)

    NPC = CH // L

    def body(i, _):
        r = i // NPC
        q = i % NPC
        idx = colv[r, pl.ds(q * L, L)]
        vals = eav[r, pl.ds(q * L, L)]
        plsc.addupdate_scatter(degv, [idx], vals)
        return 0

    lax.fori_loop(0, DNCH * NPC, body, 0)
    pltpu.sync_copy(degv, pdeg_hbm.at[pl.ds(wid * NNP, NNP)])


def _sc_agg_body(y_hbm, row_hbm, col_hbm, ea_hbm, out_hbm, rowv, colv, eav,
                 bufs, aggs, gsems, ssems):
    cid = lax.axis_index("c")
    sid = lax.axis_index("s")
    wid = sid * NC + cid
    pltpu.sync_copy(row_hbm.at[wid], rowv)
    pltpu.sync_copy(col_hbm.at[wid], colv)
    pltpu.sync_copy(ea_hbm.at[wid], eav)

    def zbuf(i, _):
        r = i // (D // L)
        q = i % (D // L)
        bufs[0][r, pl.ds(q * L, L)] = jnp.zeros((L,), jnp.float32)
        return 0

    lax.fori_loop(0, SUB * (D // L), zbuf, 0)

    NZF = NPS // SUB
    NZR = NPS % SUB

    def zagg(k, _):
        pltpu.sync_copy(bufs[0], aggs.at[pl.ds(sid * NPS + k * SUB, SUB)])
        return 0

    lax.fori_loop(0, NZF, zagg, 0)
    pltpu.sync_copy(bufs[0].at[pl.ds(0, NZR)],
                    aggs.at[pl.ds(sid * NPS + NZF * SUB, NZR)])

    @pl.when(sid == NS - 1)
    def _():
        pltpu.sync_copy(bufs[0].at[pl.ds(0, NREM)],
                        aggs.at[pl.ds(NS * NPS, NREM)])

    plsc.subcore_barrier()

    def _scale(buf, j):
        def sgroup(g0, _):
            ev = eav[j, pl.ds(g0 * L, L)]
            svs = [jnp.full((L,), ev[t], jnp.float32) for t in range(L)]
            for t in range(L):
                r = g0 * L + t
                for q in range(D // L):
                    buf[r, pl.ds(q * L, L)] = buf[r, pl.ds(q * L, L)] * svs[t]
            return 0

        lax.fori_loop(0, SUB // L, sgroup, 0)

    def _gather(b, j):
        pltpu.async_copy(y_hbm.at[rowv.at[j]], bufs[b], gsems[b])

    def _gwait(b, j):
        pltpu.make_async_copy(y_hbm.at[rowv.at[j]], bufs[b], gsems[b]).wait()

    def _scat(b, j):
        pltpu.async_copy(bufs[b], aggs.at[colv.at[j]], ssems[b], add=True)

    def _swait(b, j):
        pltpu.make_async_copy(bufs[b], aggs.at[colv.at[j]], ssems[b]).wait()

    for b in range(2):
        _gather(b, b)

    def slot_body(k, _):
        j0 = 4 * k
        for b in range(NBUF):
            j = j0 + b

            @pl.when(j >= 2)
            def _():
                _swait((b - 2) % NBUF, j - 2)

            @pl.when(j + 2 < NSLOT)
            def _():
                _gather((b + 2) % NBUF, j + 2)

            _gwait(b, j)
            _scale(bufs[b], j)
            _scat(b, j)
        return 0

    lax.fori_loop(0, NSLOT // NBUF, slot_body, 0)
    for j in range(NSLOT - 2, NSLOT):
        _swait(j % NBUF, j)
    plsc.subcore_barrier()
    pltpu.sync_copy(aggs.at[pl.ds(sid * NPS, NPS)],
                    out_hbm.at[pl.ds(cid * NN + sid * NPS, NPS)])

    @pl.when(sid == NS - 1)
    def _():
        pltpu.sync_copy(aggs.at[pl.ds(NS * NPS, NREM)],
                        out_hbm.at[pl.ds(cid * NN + NS * NPS, NREM)])


@functools.lru_cache(maxsize=1)
def _sc_kernels():
    mesh = plsc.VectorSubcoreMesh(core_axis_name="c", subcore_axis_name="s",
                                  num_cores=NC, num_subcores=NS)
    params = pltpu.CompilerParams(needs_layout_passes=False)
    sc_deg = pl.kernel(
        _sc_deg_body,
        out_type=jax.ShapeDtypeStruct((NW * NNP,), jnp.float32),
        mesh=mesh,
        compiler_params=params,
        scratch_types=[
            pltpu.VMEM((DNCH, CH), jnp.int32),
            pltpu.VMEM((DNCH, CH), jnp.float32),
            pltpu.VMEM((NNP,), jnp.float32),
        ],
    )
    sc_agg = pl.kernel(
        _sc_agg_body,
        out_type=jax.ShapeDtypeStruct((NC * NN, D), jnp.float32),
        mesh=mesh,
        compiler_params=pltpu.CompilerParams(needs_layout_passes=False,
                                             use_tc_tiling_on_sc=False),
        scratch_types=[
            pltpu.VMEM((NSLOT, SUB), jnp.int32),
            pltpu.VMEM((NSLOT, SUB), jnp.int32),
            pltpu.VMEM((NSLOT, SUB), jnp.float32),
            tuple(pltpu.VMEM((SUB, D), jnp.float32) for _ in range(NBUF)),
            pltpu.VMEM_SHARED((NN, D), jnp.float32),
            tuple(pltpu.SemaphoreType.DMA for _ in range(NBUF)),
            tuple(pltpu.SemaphoreType.DMA for _ in range(NBUF)),
        ],
    )
    return sc_deg, sc_agg


def _tc_dinv_body(pdeg_ref, dinv_ref):
    deg = jnp.sum(pdeg_ref[...], axis=0, keepdims=True)
    safe = jnp.where(deg > 0, deg, 1.0)
    dinv_ref[...] = jnp.where(deg > 0, lax.rsqrt(safe), 0.0)


_tc_dinv = pl.pallas_call(
    _tc_dinv_body, out_shape=jax.ShapeDtypeStruct((1, NNP), jnp.float32))


def _tc_y1_body(x_ref, dinv_ref, w_ref, y_ref):
    y_ref[...] = jnp.dot(x_ref[...] * dinv_ref[...], w_ref[...],
                         preferred_element_type=jnp.float32)


_tc_y1 = pl.pallas_call(
    _tc_y1_body, out_shape=jax.ShapeDtypeStruct((NN, D), jnp.float32))


def _combine(agg_ref):
    return agg_ref[0:NN, :] + agg_ref[NN:2 * NN, :]


def _tc_mid_body(agg_ref, dinv_ref, b1_ref, w2_ref, y2_ref):
    h1 = jax.nn.relu(_combine(agg_ref) * dinv_ref[...] + b1_ref[...])
    y2_ref[...] = jnp.dot(h1 * dinv_ref[...], w2_ref[...],
                          preferred_element_type=jnp.float32)


_tc_mid = pl.pallas_call(
    _tc_mid_body, out_shape=jax.ShapeDtypeStruct((NN, D), jnp.float32))


def _tc_h2_body(agg_ref, dinv_ref, b2_ref, h2_ref):
    h2_ref[...] = jax.nn.relu(_combine(agg_ref) * dinv_ref[...] + b2_ref[...])


_tc_h2 = pl.pallas_call(
    _tc_h2_body, out_shape=jax.ShapeDtypeStruct((NN, D), jnp.float32))


def _tc_lstm_body(h2_ref, wf_ref, wb_ref, wr_ref, bc_ref,
                  wl_ref, bl_ref, out_ref, gf_s, gb_s, hcat):
    h2 = h2_ref[...]
    gf_s[...] = jnp.dot(h2, wf_ref[...],
                        preferred_element_type=jnp.float32) + bc_ref[...]
    gb_s[...] = jnp.dot(h2, wb_ref[...], preferred_element_type=jnp.float32)

    def step(t, carry):
        h, c = carry
        g = (gf_s[pl.ds(t, 1), :] + gb_s[pl.ds(NN - 1 - t, 1), :] +
             jnp.dot(h.astype(jnp.bfloat16), wr_ref[...],
                     preferred_element_type=jnp.float32))
        sg = jax.nn.sigmoid(g[:, 0:384])
        th = jnp.tanh(g[:, 384:512])
        c = sg[:, 128:256] * c + sg[:, 0:128] * th
        h = sg[:, 256:384] * jnp.tanh(c)
        hcat[pl.ds(t, 1), 0:LH] = h[:, 0:LH]
        hcat[pl.ds(NN - 1 - t, 1), LH:2 * LH] = h[:, LH:2 * LH]
        return h, c

    z = jnp.zeros((1, 2 * LH), jnp.float32)
    lax.fori_loop(0, NN, step, (z, z))
    p = jnp.dot(hcat[...], wl_ref[...],
                preferred_element_type=jnp.float32) + bl_ref[...]
    out_ref[...] = jax.nn.sigmoid(p)


_tc_lstm = pl.pallas_call(
    _tc_lstm_body,
    out_shape=jax.ShapeDtypeStruct((NN, 1), jnp.float32),
    scratch_shapes=[
        pltpu.VMEM((NN, 512), jnp.float32),
        pltpu.VMEM((NN, 512), jnp.float32),
        pltpu.VMEM((NN, 2 * LH), jnp.float32),
    ],
    compiler_params=pltpu.CompilerParams(vmem_limit_bytes=62 * 1024 * 1024),
)


def _lstm_weights(Wih_f, Whh_f, bih_f, bhh_f, Wih_b, Whh_b, bih_b, bhh_b):
    def slots(WT):
        return WT[:, 0:LH], WT[:, LH:2 * LH], WT[:, 3 * LH:4 * LH], WT[:, 2 * LH:3 * LH]

    def place(WT, off):
        i_, f_, o_, g_ = slots(WT)
        out = jnp.zeros((WT.shape[0], 512), WT.dtype)
        out = out.at[:, 0 + off:LH + off].set(i_)
        out = out.at[:, 128 + off:128 + LH + off].set(f_)
        out = out.at[:, 256 + off:256 + LH + off].set(o_)
        out = out.at[:, 384 + off:384 + LH + off].set(g_)
        return out

    WF = place(Wih_f.T, 0)
    WB = place(Wih_b.T, LH)
    WR = jnp.zeros((2 * LH, 512), Wih_f.dtype)
    WR = WR + place(jnp.concatenate([Whh_f.T, jnp.zeros_like(Whh_f.T)], 0), 0)
    WR = WR + place(jnp.concatenate([jnp.zeros_like(Whh_b.T), Whh_b.T], 0), LH)
    bc = (place((bih_f + bhh_f).reshape(1, G4), 0) +
          place((bih_b + bhh_b).reshape(1, G4), LH))
    return WF, WB, WR, bc


def kernel(x, edge_index, edge_attr, W1, b1, W2, b2, Wih_f, Whh_f, bih_f,
           bhh_f, Wih_b, Whh_b, bih_b, bhh_b, Wl, bl):
    _sc_deg, _sc_agg = _sc_kernels()
    row = edge_index[0]
    col = edge_index[1]
    dpad = NW * DEPW - NE
    colp_d = jnp.concatenate([col, jnp.zeros((dpad,), jnp.int32)])
    colp_d = colp_d.reshape(NW, DNCH, CH)
    eap_d = jnp.concatenate([edge_attr, jnp.zeros((dpad,), jnp.float32)])
    eap_d = eap_d.reshape(NW, DNCH, CH)

    apad = NW * AEPW - NE
    zi = jnp.zeros((apad,), jnp.int32)
    rowp = jnp.concatenate([row, zi]).reshape(NW, NSLOT, SUB)
    colp = jnp.concatenate([col, zi]).reshape(NW, NSLOT, SUB)
    eap = jnp.concatenate([edge_attr, jnp.zeros((apad,), jnp.float32)])
    eap = eap.reshape(NW, NSLOT, SUB)

    pdeg = _sc_deg(colp_d, eap_d).reshape(NW, NNP)
    dinv = _tc_dinv(pdeg)[:, :NN].reshape(NN, 1)

    y1 = _tc_y1(x, dinv, W1)
    agg1 = _sc_agg(y1, rowp, colp, eap)
    y2 = _tc_mid(agg1, dinv, b1.reshape(1, D), W2)
    agg2 = _sc_agg(y2, rowp, colp, eap)

    WF, WB, WR, bc = _lstm_weights(Wih_f, Whh_f, bih_f, bhh_f, Wih_b, Whh_b,
                                   bih_b, bhh_b)
    h2 = _tc_h2(agg2, dinv, b2.reshape(1, D))
    out = _tc_lstm(h2, WF, WB, WR.astype(jnp.bfloat16), bc, Wl,
                   bl.reshape(1, 1))
    return out.reshape(1, NN)
